# baseline (speedup 1.0000x reference)
import functools
import jax
import jax.numpy as jnp
from jax import lax
from jax.experimental import pallas as pl
from jax.experimental.pallas import tpu as pltpu
from jax.experimental.pallas import tpu_sc as plsc

N_NODES = 10000
N_PAD = 10112
T_STEPS = 8
TP = T_STEPS // 2
F_IN = 128
HG = 64
HW = 2 * HG
HT = 128
E_EDGES = 320000
NUM_CORES = 2
NUM_SUBCORES = 16
NW = NUM_CORES * NUM_SUBCORES
CHUNK = 128
N_CHUNKS = 80
HALF = N_CHUNKS // 2
EDGE_SLAB = N_CHUNKS * CHUNK
E_PAD = NW * EDGE_SLAB
ZROWS = N_PAD // NUM_SUBCORES
LROWS = N_NODES - (NUM_SUBCORES - 1) * ZROWS
RBLK = 2000
NB = N_NODES // RBLK


def _tc1_body(x_ref, wl_ref, wr_ref, p_ref, r_ref):
    x0 = x_ref[0]
    x1 = x_ref[1]
    p_ref[0] = jnp.concatenate(
        [jnp.dot(x0, wl_ref[...], preferred_element_type=jnp.float32, precision=jax.lax.Precision.HIGHEST),
         jnp.dot(x1, wl_ref[...], preferred_element_type=jnp.float32, precision=jax.lax.Precision.HIGHEST)],
        axis=1)
    r_ref[0] = jnp.concatenate(
        [jnp.dot(x0, wr_ref[...], preferred_element_type=jnp.float32, precision=jax.lax.Precision.HIGHEST),
         jnp.dot(x1, wr_ref[...], preferred_element_type=jnp.float32, precision=jax.lax.Precision.HIGHEST)],
        axis=1)


def _tc2_body(a_ref, cnt_ref, r1_ref, bl1_ref, wl2t_ref, wr2t_ref,
              p2_ref, r2_ref):
    a = a_ref[0, 0] + a_ref[1, 0]
    c = cnt_ref[0] + cnt_ref[1]
    rc = 1.0 / jnp.maximum(c, 1.0)
    am = a * rc
    p2s, r2s = [], []
    for k in range(2):
        h = jnp.maximum(am[:, k * HG:(k + 1) * HG] + bl1_ref[...]
                        + r1_ref[0, :, k * HG:(k + 1) * HG], 0.0)
        p2s.append(jnp.dot(h, wl2t_ref[...],
                           preferred_element_type=jnp.float32, precision=jax.lax.Precision.HIGHEST))
        r2s.append(jnp.dot(h, wr2t_ref[...],
                           preferred_element_type=jnp.float32, precision=jax.lax.Precision.HIGHEST))
    p2_ref[0] = jnp.concatenate(p2s, axis=1)
    r2_ref[0] = jnp.concatenate(r2s, axis=1)


def _tc3_body(a_ref, cnt_ref, r2_ref, bl2_ref, wiht_ref, whht_ref, bih_ref,
              bhh_ref, wh1t_ref, bh1_ref, wh2t_ref, bh2_ref, y_ref, hs_ref):
    p = pl.program_id(0)
    j = pl.program_id(1)
    a = a_ref[0, 0] + a_ref[1, 0]
    c = cnt_ref[0] + cnt_ref[1]
    rc = 1.0 / jnp.maximum(c, 1.0)
    am = a * rc
    for k in range(2):
        h2 = jnp.maximum(am[:, k * HG:(k + 1) * HG] + bl2_ref[...]
                         + r2_ref[0, :, k * HG:(k + 1) * HG], 0.0)
        part = jnp.sum(h2, axis=0, keepdims=True) * (1.0 / N_NODES)
        idx = pl.ds(2 * p + k, 1)
        prev = jnp.where(j == 0, jnp.zeros((1, HG), jnp.float32),
                         hs_ref[idx, :])
        hs_ref[idx, :] = prev + part

    @pl.when((p == TP - 1) & (j == NB - 1))
    def _gru():
        gi_all = jnp.dot(hs_ref[...], wiht_ref[...],
                         preferred_element_type=jnp.float32, precision=jax.lax.Precision.HIGHEST) + bih_ref[...]
        hstate = jnp.zeros((1, HT), jnp.float32)
        for tt in range(T_STEPS):
            gh = jnp.dot(hstate, whht_ref[...],
                         preferred_element_type=jnp.float32, precision=jax.lax.Precision.HIGHEST) + bhh_ref[...]
            gi = gi_all[tt:tt + 1, :]
            r = jax.nn.sigmoid(gi[:, 0:HT] + gh[:, 0:HT])
            z = jax.nn.sigmoid(gi[:, HT:2 * HT] + gh[:, HT:2 * HT])
            n = jnp.tanh(gi[:, 2 * HT:] + r * gh[:, 2 * HT:])
            hstate = (1.0 - z) * n + z * hstate
        hid = jnp.maximum(
            jnp.dot(hstate, wh1t_ref[...], preferred_element_type=jnp.float32, precision=jax.lax.Precision.HIGHEST)
            + bh1_ref[...], 0.0)
        y_ref[...] = jnp.dot(hid, wh2t_ref[...],
                             preferred_element_type=jnp.float32, precision=jax.lax.Precision.HIGHEST) + bh2_ref[...]


def _agg_body(compute_cnt, *refs):
    if compute_cnt:
        (table, srcp, dstp, zeros, ones, a_out, cnt_out,
         src_v, dst_v, buf0, buf1, a_sh, sem0, sem1) = refs
    else:
        (table, srcp, dstp, zeros, ones, a_out,
         src_v, dst_v, buf0, buf1, a_sh, sem0, sem1) = refs
    c = lax.axis_index("c")
    s = lax.axis_index("s")
    wid = c * NUM_SUBCORES + s

    def stage_idx(half):
        pltpu.sync_copy(srcp.at[wid].at[pl.ds(half * HALF, HALF)], src_v)
        pltpu.sync_copy(dstp.at[wid].at[pl.ds(half * HALF, HALF)], dst_v)

    def zero_acc():
        pltpu.sync_copy(zeros, a_sh.at[pl.ds(s * ZROWS, ZROWS)])

    def copy_out(dst_view):
        @pl.when(s < NUM_SUBCORES - 1)
        def _full():
            pltpu.sync_copy(a_sh.at[pl.ds(s * ZROWS, ZROWS)],
                            dst_view.at[pl.ds(s * ZROWS, ZROWS)])

        @pl.when(s == NUM_SUBCORES - 1)
        def _last():
            pltpu.sync_copy(a_sh.at[pl.ds(s * ZROWS, LROWS)],
                            dst_view.at[pl.ds(s * ZROWS, LROWS)])

    if compute_cnt:
        pltpu.sync_copy(ones, buf0)
        zero_acc()
        plsc.subcore_barrier()
        for half in range(2):
            stage_idx(half)

            def cnt_step(j, carry):
                pltpu.sync_copy(buf0, a_sh.at[dst_v.at[j]], add=True)
                return carry

            lax.fori_loop(0, HALF, cnt_step, 0)
        plsc.subcore_barrier()
        copy_out(cnt_out.at[c])
        plsc.subcore_barrier()

    for p in range(TP):
        zero_acc()
        plsc.subcore_barrier()
        for half in range(2):
            stage_idx(half)
            pltpu.async_copy(table.at[p].at[src_v.at[0]], buf0, sem0)

            def agg_step(jj, carry):
                a = 2 * jj
                pltpu.async_copy(table.at[p].at[src_v.at[a + 1]], buf1, sem1)
                pltpu.make_async_copy(
                    table.at[p].at[src_v.at[a]], buf0, sem0).wait()
                pltpu.sync_copy(buf0, a_sh.at[dst_v.at[a]], add=True)

                @pl.when(a + 2 < HALF)
                def _():
                    pltpu.async_copy(
                        table.at[p].at[src_v.at[a + 2]], buf0, sem0)

                pltpu.make_async_copy(
                    table.at[p].at[src_v.at[a + 1]], buf1, sem1).wait()
                pltpu.sync_copy(buf1, a_sh.at[dst_v.at[a + 1]], add=True)
                return carry

            lax.fori_loop(0, HALF // 2, agg_step, 0)
        plsc.subcore_barrier()
        copy_out(a_out.at[c, p])
        plsc.subcore_barrier()


def _make_sc_agg(compute_cnt):
    mesh = plsc.VectorSubcoreMesh(core_axis_name="c", subcore_axis_name="s")
    out_type = [jax.ShapeDtypeStruct((NUM_CORES, TP, N_NODES, HW),
                                     jnp.float32)]
    if compute_cnt:
        out_type.append(jax.ShapeDtypeStruct((NUM_CORES, N_NODES, HW),
                                             jnp.float32))
    return pl.kernel(
        functools.partial(_agg_body, compute_cnt),
        mesh=mesh,
        out_type=out_type,
        scratch_types=[
            pltpu.VMEM((HALF, CHUNK), jnp.int32),
            pltpu.VMEM((HALF, CHUNK), jnp.int32),
            pltpu.VMEM((CHUNK, HW), jnp.float32),
            pltpu.VMEM((CHUNK, HW), jnp.float32),
            pltpu.VMEM_SHARED((N_PAD, HW), jnp.float32),
            pltpu.SemaphoreType.DMA,
            pltpu.SemaphoreType.DMA,
        ],
    )


@jax.jit
def kernel(x_seq, edge_index, Wl1, bl1, Wr1, Wl2, bl2, Wr2, Wih, Whh, bih,
           bhh, Wh1, bh1, Wh2, bh2):
    X = x_seq.reshape(T_STEPS, N_NODES, F_IN)
    src = edge_index[0].astype(jnp.int32)
    dst = edge_index[1].astype(jnp.int32)
    pad = E_PAD - E_EDGES
    srcp = jnp.concatenate([src, jnp.zeros((pad,), jnp.int32)])
    dummy_dst = N_NODES + (jnp.arange(pad, dtype=jnp.int32)
                           % (N_PAD - N_NODES))
    dstp = jnp.concatenate([dst, dummy_dst])
    srcp = srcp.reshape(NW, N_CHUNKS, CHUNK)
    dstp = dstp.reshape(NW, N_CHUNKS, CHUNK)
    zeros_blk = jnp.zeros((ZROWS, HW), jnp.float32)
    ones_blk = jnp.ones((CHUNK, HW), jnp.float32)

    blk = 2000
    nb = N_NODES // blk
    P1p, R1p = pl.pallas_call(
        _tc1_body,
        grid=(TP, nb),
        in_specs=[
            pl.BlockSpec((2, blk, F_IN), lambda p, i: (p, i, 0)),
            pl.BlockSpec((F_IN, HG), lambda p, i: (0, 0)),
            pl.BlockSpec((F_IN, HG), lambda p, i: (0, 0)),
        ],
        out_specs=[
            pl.BlockSpec((1, blk, HW), lambda p, i: (p, i, 0)),
            pl.BlockSpec((1, blk, HW), lambda p, i: (p, i, 0)),
        ],
        out_shape=[
            jax.ShapeDtypeStruct((TP, N_NODES, HW), jnp.float32),
            jax.ShapeDtypeStruct((TP, N_NODES, HW), jnp.float32),
        ],
    )(X, Wl1.T, Wr1.T)

    A1p, cntp = _make_sc_agg(True)(P1p, srcp, dstp, zeros_blk, ones_blk)

    P2p, R2p = pl.pallas_call(
        _tc2_body,
        grid=(TP, NB),
        in_specs=[
            pl.BlockSpec((NUM_CORES, 1, RBLK, HW), lambda p, j: (0, p, j, 0)),
            pl.BlockSpec((NUM_CORES, RBLK, HW), lambda p, j: (0, j, 0)),
            pl.BlockSpec((1, RBLK, HW), lambda p, j: (p, j, 0)),
            pl.BlockSpec((1, HG), lambda p, j: (0, 0)),
            pl.BlockSpec((HG, HG), lambda p, j: (0, 0)),
            pl.BlockSpec((HG, HG), lambda p, j: (0, 0)),
        ],
        out_specs=[
            pl.BlockSpec((1, RBLK, HW), lambda p, j: (p, j, 0)),
            pl.BlockSpec((1, RBLK, HW), lambda p, j: (p, j, 0)),
        ],
        out_shape=[
            jax.ShapeDtypeStruct((TP, N_NODES, HW), jnp.float32),
            jax.ShapeDtypeStruct((TP, N_NODES, HW), jnp.float32),
        ],
    )(A1p, cntp, R1p, bl1.reshape(1, HG), Wl2.T, Wr2.T)

    (A2p,) = _make_sc_agg(False)(P2p, srcp, dstp, zeros_blk, ones_blk)

    y2 = pl.pallas_call(
        _tc3_body,
        grid=(TP, NB),
        in_specs=[
            pl.BlockSpec((NUM_CORES, 1, RBLK, HW), lambda p, j: (0, p, j, 0)),
            pl.BlockSpec((NUM_CORES, RBLK, HW), lambda p, j: (0, j, 0)),
            pl.BlockSpec((1, RBLK, HW), lambda p, j: (p, j, 0)),
            pl.BlockSpec((1, HG), lambda p, j: (0, 0)),
            pl.BlockSpec((HG, 3 * HT), lambda p, j: (0, 0)),
            pl.BlockSpec((HT, 3 * HT), lambda p, j: (0, 0)),
            pl.BlockSpec((1, 3 * HT), lambda p, j: (0, 0)),
            pl.BlockSpec((1, 3 * HT), lambda p, j: (0, 0)),
            pl.BlockSpec((HT, HG), lambda p, j: (0, 0)),
            pl.BlockSpec((1, HG), lambda p, j: (0, 0)),
            pl.BlockSpec((HG, 1), lambda p, j: (0, 0)),
            pl.BlockSpec((1, 1), lambda p, j: (0, 0)),
        ],
        out_specs=pl.BlockSpec((1, 1), lambda p, j: (0, 0)),
        out_shape=jax.ShapeDtypeStruct((1, 1), jnp.float32),
        scratch_shapes=[pltpu.VMEM((T_STEPS, HG), jnp.float32)],
    )(A2p, cntp, R2p, bl2.reshape(1, HG), Wih.T, Whh.T,
      bih.reshape(1, 3 * HT), bhh.reshape(1, 3 * HT), Wh1.T,
      bh1.reshape(1, HG), Wh2.T, bh2.reshape(1, 1))

    return y2.reshape(1)


# speedup vs baseline: 3.2398x; 3.2398x over previous
import functools
import jax
import jax.numpy as jnp
from jax import lax
from jax.experimental import pallas as pl
from jax.experimental.pallas import tpu as pltpu
from jax.experimental.pallas import tpu_sc as plsc

N_NODES = 10000
N_PAD = 10112
T_STEPS = 8
TP = T_STEPS // 2
F_IN = 128
HG = 64
HW = 2 * HG
HT = 128
E_EDGES = 320000
NUM_CORES = 2
NUM_SUBCORES = 16
NW = NUM_CORES * NUM_SUBCORES
CHUNK = 128
N_CHUNKS = 80
HALF = N_CHUNKS // 2
EDGE_SLAB = N_CHUNKS * CHUNK
E_PAD = NW * EDGE_SLAB
ZROWS = N_PAD // NUM_SUBCORES
LROWS = N_NODES - (NUM_SUBCORES - 1) * ZROWS
RBLK = 2000
NB = N_NODES // RBLK


def _tc1_body(x_ref, wl_ref, wr_ref, p_ref, r_ref):
    x0 = x_ref[0]
    x1 = x_ref[1]
    p_ref[0] = jnp.concatenate(
        [jnp.dot(x0, wl_ref[...], preferred_element_type=jnp.float32, precision=jax.lax.Precision.HIGHEST),
         jnp.dot(x1, wl_ref[...], preferred_element_type=jnp.float32, precision=jax.lax.Precision.HIGHEST)],
        axis=1)
    r_ref[0] = jnp.concatenate(
        [jnp.dot(x0, wr_ref[...], preferred_element_type=jnp.float32, precision=jax.lax.Precision.HIGHEST),
         jnp.dot(x1, wr_ref[...], preferred_element_type=jnp.float32, precision=jax.lax.Precision.HIGHEST)],
        axis=1)


def _tc2_body(a_ref, cnt_ref, r1_ref, bl1_ref, wl2t_ref, wr2t_ref,
              p2_ref, r2_ref):
    a = a_ref[0, 0] + a_ref[1, 0]
    c = cnt_ref[0] + cnt_ref[1]
    rc = 1.0 / jnp.maximum(c, 1.0)
    am = a * rc
    p2s, r2s = [], []
    for k in range(2):
        h = jnp.maximum(am[:, k * HG:(k + 1) * HG] + bl1_ref[...]
                        + r1_ref[0, :, k * HG:(k + 1) * HG], 0.0)
        p2s.append(jnp.dot(h, wl2t_ref[...],
                           preferred_element_type=jnp.float32, precision=jax.lax.Precision.HIGHEST))
        r2s.append(jnp.dot(h, wr2t_ref[...],
                           preferred_element_type=jnp.float32, precision=jax.lax.Precision.HIGHEST))
    p2_ref[0] = jnp.concatenate(p2s, axis=1)
    r2_ref[0] = jnp.concatenate(r2s, axis=1)


def _tc3_body(a_ref, cnt_ref, r2_ref, bl2_ref, wiht_ref, whht_ref, bih_ref,
              bhh_ref, wh1t_ref, bh1_ref, wh2t_ref, bh2_ref, y_ref, hs_ref):
    p = pl.program_id(0)
    j = pl.program_id(1)
    a = a_ref[0, 0] + a_ref[1, 0]
    c = cnt_ref[0] + cnt_ref[1]
    rc = 1.0 / jnp.maximum(c, 1.0)
    am = a * rc
    for k in range(2):
        h2 = jnp.maximum(am[:, k * HG:(k + 1) * HG] + bl2_ref[...]
                         + r2_ref[0, :, k * HG:(k + 1) * HG], 0.0)
        part = jnp.sum(h2, axis=0, keepdims=True) * (1.0 / N_NODES)
        idx = pl.ds(2 * p + k, 1)
        prev = jnp.where(j == 0, jnp.zeros((1, HG), jnp.float32),
                         hs_ref[idx, :])
        hs_ref[idx, :] = prev + part

    @pl.when((p == TP - 1) & (j == NB - 1))
    def _gru():
        gi_all = jnp.dot(hs_ref[...], wiht_ref[...],
                         preferred_element_type=jnp.float32, precision=jax.lax.Precision.HIGHEST) + bih_ref[...]
        hstate = jnp.zeros((1, HT), jnp.float32)
        for tt in range(T_STEPS):
            gh = jnp.dot(hstate, whht_ref[...],
                         preferred_element_type=jnp.float32, precision=jax.lax.Precision.HIGHEST) + bhh_ref[...]
            gi = gi_all[tt:tt + 1, :]
            r = jax.nn.sigmoid(gi[:, 0:HT] + gh[:, 0:HT])
            z = jax.nn.sigmoid(gi[:, HT:2 * HT] + gh[:, HT:2 * HT])
            n = jnp.tanh(gi[:, 2 * HT:] + r * gh[:, 2 * HT:])
            hstate = (1.0 - z) * n + z * hstate
        hid = jnp.maximum(
            jnp.dot(hstate, wh1t_ref[...], preferred_element_type=jnp.float32, precision=jax.lax.Precision.HIGHEST)
            + bh1_ref[...], 0.0)
        y_ref[...] = jnp.dot(hid, wh2t_ref[...],
                             preferred_element_type=jnp.float32, precision=jax.lax.Precision.HIGHEST) + bh2_ref[...]


def _agg_body(compute_cnt, *refs):
    if compute_cnt:
        (table, srcp, dstp, zeros, ones, a_out, cnt_out,
         src_v, dst_v, buf0, buf1, a_sh, sem0, sem1) = refs
    else:
        (table, srcp, dstp, zeros, ones, a_out,
         src_v, dst_v, buf0, buf1, a_sh, sem0, sem1) = refs
    c = lax.axis_index("c")
    s = lax.axis_index("s")
    wid = c * NUM_SUBCORES + s

    def stage_idx(half):
        pltpu.sync_copy(srcp.at[wid].at[pl.ds(half * HALF, HALF)], src_v)
        pltpu.sync_copy(dstp.at[wid].at[pl.ds(half * HALF, HALF)], dst_v)

    def zero_acc():
        pltpu.sync_copy(zeros, a_sh.at[pl.ds(s * ZROWS, ZROWS)])

    def copy_out(dst_view):
        @pl.when(s < NUM_SUBCORES - 1)
        def _full():
            pltpu.sync_copy(a_sh.at[pl.ds(s * ZROWS, ZROWS)],
                            dst_view.at[pl.ds(s * ZROWS, ZROWS)])

        @pl.when(s == NUM_SUBCORES - 1)
        def _last():
            pltpu.sync_copy(a_sh.at[pl.ds(s * ZROWS, LROWS)],
                            dst_view.at[pl.ds(s * ZROWS, LROWS)])

    if compute_cnt:
        pltpu.sync_copy(ones, buf0)
        zero_acc()
        plsc.subcore_barrier()
        for half in range(2):
            stage_idx(half)

            def cnt_step(j, carry):
                pltpu.sync_copy(buf0, a_sh.at[dst_v.at[j]], add=True)
                return carry

            lax.fori_loop(0, HALF, cnt_step, 0)
        plsc.subcore_barrier()
        copy_out(cnt_out.at[c])
        plsc.subcore_barrier()

    for p in range(TP):
        zero_acc()
        plsc.subcore_barrier()
        for half in range(2):
            stage_idx(half)
            pltpu.async_copy(table.at[p].at[src_v.at[0]], buf0, sem0)

            def agg_step(jj, carry):
                a = 2 * jj
                pltpu.async_copy(table.at[p].at[src_v.at[a + 1]], buf1, sem1)
                pltpu.make_async_copy(
                    table.at[p].at[src_v.at[a]], buf0, sem0).wait()
                pltpu.sync_copy(buf0, a_sh.at[dst_v.at[a]], add=True)

                @pl.when(a + 2 < HALF)
                def _():
                    pltpu.async_copy(
                        table.at[p].at[src_v.at[a + 2]], buf0, sem0)

                pltpu.make_async_copy(
                    table.at[p].at[src_v.at[a + 1]], buf1, sem1).wait()
                pltpu.sync_copy(buf1, a_sh.at[dst_v.at[a + 1]], add=True)
                return carry

            lax.fori_loop(0, HALF // 2, agg_step, 0)
        plsc.subcore_barrier()
        copy_out(a_out.at[c, p])
        plsc.subcore_barrier()


def _make_sc_agg(compute_cnt):
    mesh = plsc.VectorSubcoreMesh(core_axis_name="c", subcore_axis_name="s")
    out_type = [jax.ShapeDtypeStruct((NUM_CORES, TP, N_NODES, HW),
                                     jnp.float32)]
    if compute_cnt:
        out_type.append(jax.ShapeDtypeStruct((NUM_CORES, N_NODES, HW),
                                             jnp.float32))
    return pl.kernel(
        functools.partial(_agg_body, compute_cnt),
        mesh=mesh,
        out_type=out_type,
        scratch_types=[
            pltpu.VMEM((HALF, CHUNK), jnp.int32),
            pltpu.VMEM((HALF, CHUNK), jnp.int32),
            pltpu.VMEM((CHUNK, HW), jnp.float32),
            pltpu.VMEM((CHUNK, HW), jnp.float32),
            pltpu.VMEM_SHARED((N_PAD, HW), jnp.float32),
            pltpu.SemaphoreType.DMA,
            pltpu.SemaphoreType.DMA,
        ],
    )


@jax.jit
def kernel(x_seq, edge_index, Wl1, bl1, Wr1, Wl2, bl2, Wr2, Wih, Whh, bih,
           bhh, Wh1, bh1, Wh2, bh2):
    X = x_seq.reshape(T_STEPS, N_NODES, F_IN)
    src = edge_index[0].astype(jnp.int32)
    dst = edge_index[1].astype(jnp.int32)
    pad = E_PAD - E_EDGES
    dummy_src = jnp.arange(pad, dtype=jnp.int32) % N_NODES
    dummy_dst = N_NODES + (jnp.arange(pad, dtype=jnp.int32)
                           % (N_PAD - N_NODES))
    srcp = jnp.concatenate([src, dummy_src])
    dstp = jnp.concatenate([dst, dummy_dst])
    srcp = srcp.reshape(NW, N_CHUNKS, CHUNK)
    dstp = dstp.reshape(NW, N_CHUNKS, CHUNK)
    zeros_blk = jnp.zeros((ZROWS, HW), jnp.float32)
    ones_blk = jnp.ones((CHUNK, HW), jnp.float32)

    blk = 2000
    nb = N_NODES // blk
    P1p, R1p = pl.pallas_call(
        _tc1_body,
        grid=(TP, nb),
        in_specs=[
            pl.BlockSpec((2, blk, F_IN), lambda p, i: (p, i, 0)),
            pl.BlockSpec((F_IN, HG), lambda p, i: (0, 0)),
            pl.BlockSpec((F_IN, HG), lambda p, i: (0, 0)),
        ],
        out_specs=[
            pl.BlockSpec((1, blk, HW), lambda p, i: (p, i, 0)),
            pl.BlockSpec((1, blk, HW), lambda p, i: (p, i, 0)),
        ],
        out_shape=[
            jax.ShapeDtypeStruct((TP, N_NODES, HW), jnp.float32),
            jax.ShapeDtypeStruct((TP, N_NODES, HW), jnp.float32),
        ],
    )(X, Wl1.T, Wr1.T)

    A1p, cntp = _make_sc_agg(True)(P1p, srcp, dstp, zeros_blk, ones_blk)

    P2p, R2p = pl.pallas_call(
        _tc2_body,
        grid=(TP, NB),
        in_specs=[
            pl.BlockSpec((NUM_CORES, 1, RBLK, HW), lambda p, j: (0, p, j, 0)),
            pl.BlockSpec((NUM_CORES, RBLK, HW), lambda p, j: (0, j, 0)),
            pl.BlockSpec((1, RBLK, HW), lambda p, j: (p, j, 0)),
            pl.BlockSpec((1, HG), lambda p, j: (0, 0)),
            pl.BlockSpec((HG, HG), lambda p, j: (0, 0)),
            pl.BlockSpec((HG, HG), lambda p, j: (0, 0)),
        ],
        out_specs=[
            pl.BlockSpec((1, RBLK, HW), lambda p, j: (p, j, 0)),
            pl.BlockSpec((1, RBLK, HW), lambda p, j: (p, j, 0)),
        ],
        out_shape=[
            jax.ShapeDtypeStruct((TP, N_NODES, HW), jnp.float32),
            jax.ShapeDtypeStruct((TP, N_NODES, HW), jnp.float32),
        ],
    )(A1p, cntp, R1p, bl1.reshape(1, HG), Wl2.T, Wr2.T)

    (A2p,) = _make_sc_agg(False)(P2p, srcp, dstp, zeros_blk, ones_blk)

    y2 = pl.pallas_call(
        _tc3_body,
        grid=(TP, NB),
        in_specs=[
            pl.BlockSpec((NUM_CORES, 1, RBLK, HW), lambda p, j: (0, p, j, 0)),
            pl.BlockSpec((NUM_CORES, RBLK, HW), lambda p, j: (0, j, 0)),
            pl.BlockSpec((1, RBLK, HW), lambda p, j: (p, j, 0)),
            pl.BlockSpec((1, HG), lambda p, j: (0, 0)),
            pl.BlockSpec((HG, 3 * HT), lambda p, j: (0, 0)),
            pl.BlockSpec((HT, 3 * HT), lambda p, j: (0, 0)),
            pl.BlockSpec((1, 3 * HT), lambda p, j: (0, 0)),
            pl.BlockSpec((1, 3 * HT), lambda p, j: (0, 0)),
            pl.BlockSpec((HT, HG), lambda p, j: (0, 0)),
            pl.BlockSpec((1, HG), lambda p, j: (0, 0)),
            pl.BlockSpec((HG, 1), lambda p, j: (0, 0)),
            pl.BlockSpec((1, 1), lambda p, j: (0, 0)),
        ],
        out_specs=pl.BlockSpec((1, 1), lambda p, j: (0, 0)),
        out_shape=jax.ShapeDtypeStruct((1, 1), jnp.float32),
        scratch_shapes=[pltpu.VMEM((T_STEPS, HG), jnp.float32)],
    )(A2p, cntp, R2p, bl2.reshape(1, HG), Wih.T, Whh.T,
      bih.reshape(1, 3 * HT), bhh.reshape(1, 3 * HT), Wh1.T,
      bh1.reshape(1, HG), Wh2.T, bh2.reshape(1, 1))

    return y2.reshape(1)
